# baseline (speedup 1.0000x reference)
import functools

import jax
import jax.numpy as jnp
from jax import lax
from jax.experimental import pallas as pl
from jax.experimental.pallas import tpu as pltpu
from jax.experimental.pallas import tpu_sc as plsc

N = 10000
E = 320000
D = 128
NG = 64

NC = 2
NS = 16
NT = NC * NS
CHUNK = 128
NCHUNK = 80
NB = 2
HC = 32
NCH0 = 148
NCH1 = 8
EPAD = NT * NCHUNK * CHUNK
ACC = 10240
RPT = ACC // NS

ROWS_BLK = 1000


def _mesh():
    return plsc.VectorSubcoreMesh(
        core_axis_name="c", subcore_axis_name="s", num_cores=NC, num_subcores=NS
    )


def _sc_deg(dstc, onesD, zerosD):
    @functools.partial(
        pl.kernel,
        out_type=jax.ShapeDtypeStruct((NC, ACC, D), jnp.float32),
        mesh=_mesh(),
        scratch_types=[
            pltpu.VMEM((NCHUNK, CHUNK), jnp.int32),
            pltpu.VMEM((CHUNK, D), jnp.float32),
            pltpu.VMEM_SHARED((ACC, D), jnp.float32),
            pltpu.SemaphoreType.DMA,
        ],
    )
    def k(dstc_hbm, ones_hbm, zeros_hbm, out_hbm, idx_v, ones_v, acc_s, sem):
        c = lax.axis_index("c")
        s = lax.axis_index("s")
        wid = c * NS + s
        pltpu.sync_copy(zeros_hbm, acc_s.at[pl.ds(s * RPT, RPT)])
        pltpu.sync_copy(ones_hbm, ones_v)
        pltpu.sync_copy(dstc_hbm.at[wid], idx_v)
        plsc.subcore_barrier()

        def body(j, carry):
            pltpu.sync_copy(ones_v, acc_s.at[idx_v.at[j]], add=True)
            return carry

        lax.fori_loop(0, NCHUNK, body, 0)
        plsc.subcore_barrier()
        pltpu.sync_copy(acc_s.at[pl.ds(s * RPT, RPT)], out_hbm.at[c, pl.ds(s * RPT, RPT)])

    return k(dstc, onesD, zerosD)


def _sc_prop(g, edges0, edges1, zerosD):
    @functools.partial(
        pl.kernel,
        out_type=jax.ShapeDtypeStruct((NC, ACC, D), jnp.float32),
        mesh=_mesh(),
        scratch_types=[
            pltpu.VMEM((HC, CHUNK), jnp.int32),
            pltpu.VMEM((HC, CHUNK), jnp.int32),
            [pltpu.VMEM((CHUNK, D), jnp.float32)] * NB,
            pltpu.VMEM_SHARED((ACC, D), jnp.float32),
            pltpu.SemaphoreType.DMA,
        ],
    )
    def k(g0_hbm, g1_hbm, src0_hbm, dst0_hbm, src1_hbm, dst1_hbm, zeros_hbm, out_hbm,
          sidx, didx, rows, acc_s, gsem):
        c = lax.axis_index("c")
        s = lax.axis_index("s")
        pltpu.sync_copy(zeros_hbm, acc_s.at[pl.ds(s * RPT, RPT)])
        plsc.subcore_barrier()

        def slab(g_hbm, srcc_hbm, dstc_hbm, base, hc):
            pltpu.sync_copy(srcc_hbm.at[s, pl.ds(base, hc)], sidx.at[pl.ds(0, hc)])
            pltpu.sync_copy(dstc_hbm.at[s, pl.ds(base, hc)], didx.at[pl.ds(0, hc)])
            nprime = min(NB, hc)
            for b in range(nprime):
                pltpu.async_copy(g_hbm.at[sidx.at[b]], rows[b], gsem)

            def outer(jo, c2):
                jb = jo * NB
                for b in range(NB):
                    j = jb + b
                    pltpu.make_async_copy(g_hbm.at[sidx.at[j]], rows[b], gsem).wait()
                    pltpu.sync_copy(rows[b], acc_s.at[didx.at[j]], add=True)
                    nxt = j + NB

                    @pl.when(nxt < hc)
                    def _():
                        pltpu.async_copy(g_hbm.at[sidx.at[nxt]], rows[b], gsem)

                return c2

            lax.fori_loop(0, hc // NB, outer, 0)

        def run(g_hbm, srcc_hbm, dstc_hbm, nch):
            done = 0
            while done < nch:
                hc = min(HC, nch - done)
                slab(g_hbm, srcc_hbm, dstc_hbm, done, hc)
                done += hc

        @pl.when(c == 0)
        def _():
            run(g0_hbm, src0_hbm, dst0_hbm, NCH0)

        @pl.when(c == 1)
        def _():
            run(g1_hbm, src1_hbm, dst1_hbm, NCH1)

        plsc.subcore_barrier()
        pltpu.sync_copy(acc_s.at[pl.ds(s * RPT, RPT)], out_hbm.at[c, pl.ds(s * RPT, RPT)])

    return k(g[0], g[1], edges0[0], edges0[1], edges1[0], edges1[1], zerosD)


def _dis_from(deg_ref):
    return lax.rsqrt(deg_ref[0, :, 0:1] + deg_ref[1, :, 0:1] + 1.0)


_P = lax.Precision.HIGHEST


def _tc_first(x, W0, indeg2):
    def body(x_ref, w_ref, deg_ref, g_ref, g2_ref):
        dis = _dis_from(deg_ref)
        g = dis * jnp.dot(
            x_ref[...], w_ref[...], preferred_element_type=jnp.float32, precision=_P
        )
        g_ref[...] = g
        g2_ref[...] = g

    return pl.pallas_call(
        body,
        grid=(N // ROWS_BLK,),
        in_specs=[
            pl.BlockSpec((ROWS_BLK, D), lambda i: (i, 0)),
            pl.BlockSpec((D, D), lambda i: (0, 0)),
            pl.BlockSpec((NC, ROWS_BLK, D), lambda i: (0, i, 0)),
        ],
        out_specs=[
            pl.BlockSpec((ROWS_BLK, D), lambda i: (i, 0)),
            pl.BlockSpec((ROWS_BLK, D), lambda i: (i, 0)),
        ],
        out_shape=[
            jax.ShapeDtypeStruct((N, D), jnp.float32),
            jax.ShapeDtypeStruct((N, D), jnp.float32),
        ],
    )(x, W0, indeg2)


def _tc_mid(tmp, g, indeg2, b_row, W_next, batch_col):
    def body(tmp_ref, g_ref, deg_ref, b_ref, w_ref, bat_ref, gn_ref, gn2_ref, pool_ref):
        i = pl.program_id(0)
        dis = _dis_from(deg_ref)
        h = jnp.maximum(
            dis * (tmp_ref[0] + tmp_ref[1] + g_ref[...]) + b_ref[...], 0.0
        )
        oh = (bat_ref[...] == lax.broadcasted_iota(jnp.int32, (ROWS_BLK, NG), 1)).astype(
            jnp.float32
        )
        pc = lax.dot_general(
            oh, h, (((0,), (0,)), ((), ())), preferred_element_type=jnp.float32,
            precision=_P,
        )

        @pl.when(i == 0)
        def _():
            pool_ref[...] = pc

        @pl.when(i > 0)
        def _():
            pool_ref[...] += pc

        gn = dis * jnp.dot(
            h, w_ref[...], preferred_element_type=jnp.float32, precision=_P
        )
        gn_ref[...] = gn
        gn2_ref[...] = gn

    return pl.pallas_call(
        body,
        grid=(N // ROWS_BLK,),
        in_specs=[
            pl.BlockSpec((NC, ROWS_BLK, D), lambda i: (0, i, 0)),
            pl.BlockSpec((ROWS_BLK, D), lambda i: (i, 0)),
            pl.BlockSpec((NC, ROWS_BLK, D), lambda i: (0, i, 0)),
            pl.BlockSpec((1, D), lambda i: (0, 0)),
            pl.BlockSpec((D, D), lambda i: (0, 0)),
            pl.BlockSpec((ROWS_BLK, 1), lambda i: (i, 0)),
        ],
        out_specs=[
            pl.BlockSpec((ROWS_BLK, D), lambda i: (i, 0)),
            pl.BlockSpec((ROWS_BLK, D), lambda i: (i, 0)),
            pl.BlockSpec((NG, D), lambda i: (0, 0)),
        ],
        out_shape=[
            jax.ShapeDtypeStruct((N, D), jnp.float32),
            jax.ShapeDtypeStruct((N, D), jnp.float32),
            jax.ShapeDtypeStruct((NG, D), jnp.float32),
        ],
    )(tmp, g, indeg2, b_row, W_next, batch_col)


def _tc_final(tmp, g, indeg2, b_row, batch_col, lin1_W, lin1_b, lin2_W, lin2_b):
    nblk = N // ROWS_BLK

    def body(tmp_ref, g_ref, deg_ref, b_ref, bat_ref, w1_ref, b1_ref, w2_ref, b2_ref,
             pool_ref, out_ref):
        i = pl.program_id(0)
        dis = _dis_from(deg_ref)
        h = jnp.maximum(
            dis * (tmp_ref[0] + tmp_ref[1] + g_ref[...]) + b_ref[...], 0.0
        )
        oh = (bat_ref[...] == lax.broadcasted_iota(jnp.int32, (ROWS_BLK, NG), 1)).astype(
            jnp.float32
        )
        pc = lax.dot_general(
            oh, h, (((0,), (0,)), ((), ())), preferred_element_type=jnp.float32,
            precision=_P,
        )

        @pl.when(i == 0)
        def _():
            pool_ref[...] = pc

        @pl.when(i > 0)
        def _():
            pool_ref[...] += pc

        @pl.when(i == nblk - 1)
        def _():
            p = jnp.maximum(
                jnp.dot(pool_ref[...], w1_ref[...], preferred_element_type=jnp.float32,
                        precision=_P) + b1_ref[...],
                0.0,
            )
            out_ref[...] = jnp.dot(
                p, w2_ref[...], preferred_element_type=jnp.float32, precision=_P
            ) + b2_ref[...]

    return pl.pallas_call(
        body,
        grid=(nblk,),
        in_specs=[
            pl.BlockSpec((NC, ROWS_BLK, D), lambda i: (0, i, 0)),
            pl.BlockSpec((ROWS_BLK, D), lambda i: (i, 0)),
            pl.BlockSpec((NC, ROWS_BLK, D), lambda i: (0, i, 0)),
            pl.BlockSpec((1, D), lambda i: (0, 0)),
            pl.BlockSpec((ROWS_BLK, 1), lambda i: (i, 0)),
            pl.BlockSpec((D, D), lambda i: (0, 0)),
            pl.BlockSpec((1, D), lambda i: (0, 0)),
            pl.BlockSpec((D, NG), lambda i: (0, 0)),
            pl.BlockSpec((1, NG), lambda i: (0, 0)),
        ],
        out_specs=[
            pl.BlockSpec((NG, D), lambda i: (0, 0)),
            pl.BlockSpec((NG, NG), lambda i: (0, 0)),
        ],
        out_shape=[
            jax.ShapeDtypeStruct((NG, D), jnp.float32),
            jax.ShapeDtypeStruct((NG, NG), jnp.float32),
        ],
    )(tmp, g, indeg2, b_row, batch_col, lin1_W, lin1_b, lin2_W, lin2_b)


def kernel(x, edge_index, batch, W0, b0, W1, b1, W2, b2, lin1_W, lin1_b, lin2_W, lin2_b):
    src = edge_index[0]
    dst = edge_index[1]
    pad = EPAD - E
    pad_dst = N + (jnp.arange(pad, dtype=jnp.int32) % (ACC - N))
    dstp = jnp.concatenate([dst, pad_dst]).reshape(NT, NCHUNK, CHUNK)

    tot_chunks = NS * (NCH0 + NCH1)
    fpad = max(0, tot_chunks * CHUNK - E)
    fpad_dst = N + (jnp.arange(fpad, dtype=jnp.int32) % (ACC - N))
    src_f = jnp.concatenate([src, jnp.zeros((fpad,), jnp.int32)])[: tot_chunks * CHUNK]
    dst_f = jnp.concatenate([dst, fpad_dst])[: tot_chunks * CHUNK]
    src_f = src_f.reshape(tot_chunks, CHUNK)
    dst_f = dst_f.reshape(tot_chunks, CHUNK)
    n0 = NS * NCH0
    edges0 = (src_f[:n0].reshape(NS, NCH0, CHUNK), dst_f[:n0].reshape(NS, NCH0, CHUNK))
    edges1 = (src_f[n0:].reshape(NS, NCH1, CHUNK), dst_f[n0:].reshape(NS, NCH1, CHUNK))
    zerosD = jnp.zeros((RPT, D), jnp.float32)
    onesD = jnp.ones((CHUNK, D), jnp.float32)
    batch_col = batch.reshape(N, 1)
    b0r = b0.reshape(1, D)
    b1r = b1.reshape(1, D)
    b2r = b2.reshape(1, D)
    lin1_br = lin1_b.reshape(1, D)
    lin2_br = lin2_b.reshape(1, NG)

    indeg2 = _sc_deg(dstp, onesD, zerosD)
    g1 = _tc_first(x, W0, indeg2)
    tmp1 = _sc_prop(g1, edges0, edges1, zerosD)
    g2a, g2b, pooled1 = _tc_mid(tmp1, g1[0], indeg2, b0r, W1, batch_col)
    tmp2 = _sc_prop((g2a, g2b), edges0, edges1, zerosD)
    g3a, g3b, pooled2 = _tc_mid(tmp2, g2a, indeg2, b1r, W2, batch_col)
    tmp3 = _sc_prop((g3a, g3b), edges0, edges1, zerosD)
    pooled3, out = _tc_final(
        tmp3, g3a, indeg2, b2r, batch_col, lin1_W, lin1_br, lin2_W, lin2_br
    )
    return (out, pooled1, pooled2, pooled3)


# speedup vs baseline: 1.4342x; 1.4342x over previous
import functools

import jax
import jax.numpy as jnp
from jax import lax
from jax.experimental import pallas as pl
from jax.experimental.pallas import tpu as pltpu
from jax.experimental.pallas import tpu_sc as plsc

N = 10000
E = 320000
D = 128
NG = 64

NC = 2
NS = 16
NT = NC * NS
CHUNK = 128
NCHUNK = 80
NB = 2
HC = 32
NCH0 = 80
NCH1 = 80
EPAD = NT * NCHUNK * CHUNK
ACC = 10240
RPT = ACC // NS

ROWS_BLK = 1000


def _mesh():
    return plsc.VectorSubcoreMesh(
        core_axis_name="c", subcore_axis_name="s", num_cores=NC, num_subcores=NS
    )


def _sc_deg(dstc, onesD, zerosD):
    @functools.partial(
        pl.kernel,
        out_type=jax.ShapeDtypeStruct((NC, ACC, D), jnp.float32),
        mesh=_mesh(),
        scratch_types=[
            pltpu.VMEM((NCHUNK, CHUNK), jnp.int32),
            pltpu.VMEM((CHUNK, D), jnp.float32),
            pltpu.VMEM_SHARED((ACC, D), jnp.float32),
            pltpu.SemaphoreType.DMA,
        ],
    )
    def k(dstc_hbm, ones_hbm, zeros_hbm, out_hbm, idx_v, ones_v, acc_s, sem):
        c = lax.axis_index("c")
        s = lax.axis_index("s")
        wid = c * NS + s
        pltpu.sync_copy(zeros_hbm, acc_s.at[pl.ds(s * RPT, RPT)])
        pltpu.sync_copy(ones_hbm, ones_v)
        pltpu.sync_copy(dstc_hbm.at[wid], idx_v)
        plsc.subcore_barrier()

        def body(j, carry):
            pltpu.sync_copy(ones_v, acc_s.at[idx_v.at[j]], add=True)
            return carry

        lax.fori_loop(0, NCHUNK, body, 0)
        plsc.subcore_barrier()
        pltpu.sync_copy(acc_s.at[pl.ds(s * RPT, RPT)], out_hbm.at[c, pl.ds(s * RPT, RPT)])

    return k(dstc, onesD, zerosD)


def _sc_prop(g, edges0, edges1, zerosD):
    @functools.partial(
        pl.kernel,
        out_type=jax.ShapeDtypeStruct((NC, ACC, D), jnp.float32),
        mesh=_mesh(),
        scratch_types=[
            pltpu.VMEM((HC, CHUNK), jnp.int32),
            pltpu.VMEM((HC, CHUNK), jnp.int32),
            [pltpu.VMEM((CHUNK, D), jnp.float32)] * NB,
            pltpu.VMEM_SHARED((ACC, D), jnp.float32),
            pltpu.SemaphoreType.DMA,
        ],
    )
    def k(g0_hbm, g1_hbm, src0_hbm, dst0_hbm, src1_hbm, dst1_hbm, zeros_hbm, out_hbm,
          sidx, didx, rows, acc_s, gsem):
        c = lax.axis_index("c")
        s = lax.axis_index("s")
        pltpu.sync_copy(zeros_hbm, acc_s.at[pl.ds(s * RPT, RPT)])
        plsc.subcore_barrier()

        def slab(g_hbm, srcc_hbm, dstc_hbm, base, hc):
            pltpu.sync_copy(srcc_hbm.at[s, pl.ds(base, hc)], sidx.at[pl.ds(0, hc)])
            pltpu.sync_copy(dstc_hbm.at[s, pl.ds(base, hc)], didx.at[pl.ds(0, hc)])
            nprime = min(NB, hc)
            for b in range(nprime):
                pltpu.async_copy(g_hbm.at[sidx.at[b]], rows[b], gsem)

            def outer(jo, c2):
                jb = jo * NB
                for b in range(NB):
                    j = jb + b
                    pltpu.make_async_copy(g_hbm.at[sidx.at[j]], rows[b], gsem).wait()
                    pltpu.sync_copy(rows[b], acc_s.at[didx.at[j]], add=True)
                    nxt = j + NB

                    @pl.when(nxt < hc)
                    def _():
                        pltpu.async_copy(g_hbm.at[sidx.at[nxt]], rows[b], gsem)

                return c2

            lax.fori_loop(0, hc // NB, outer, 0)

        def run(g_hbm, srcc_hbm, dstc_hbm, nch):
            done = 0
            while done < nch:
                hc = min(HC, nch - done)
                slab(g_hbm, srcc_hbm, dstc_hbm, done, hc)
                done += hc

        @pl.when(c == 0)
        def _():
            run(g0_hbm, src0_hbm, dst0_hbm, NCH0)

        @pl.when(c == 1)
        def _():
            run(g1_hbm, src1_hbm, dst1_hbm, NCH1)

        plsc.subcore_barrier()
        pltpu.sync_copy(acc_s.at[pl.ds(s * RPT, RPT)], out_hbm.at[c, pl.ds(s * RPT, RPT)])

    return k(g[0], g[1], edges0[0], edges0[1], edges1[0], edges1[1], zerosD)


def _dis_from(deg_ref):
    return lax.rsqrt(deg_ref[0, :, 0:1] + deg_ref[1, :, 0:1] + 1.0)


_P = lax.Precision.HIGHEST


def _tc_first(x, W0, indeg2):
    def body(x_ref, w_ref, deg_ref, g_ref, g2_ref):
        dis = _dis_from(deg_ref)
        g = dis * jnp.dot(
            x_ref[...], w_ref[...], preferred_element_type=jnp.float32, precision=_P
        )
        g_ref[...] = g
        g2_ref[...] = g

    return pl.pallas_call(
        body,
        grid=(N // ROWS_BLK,),
        in_specs=[
            pl.BlockSpec((ROWS_BLK, D), lambda i: (i, 0)),
            pl.BlockSpec((D, D), lambda i: (0, 0)),
            pl.BlockSpec((NC, ROWS_BLK, D), lambda i: (0, i, 0)),
        ],
        out_specs=[
            pl.BlockSpec((ROWS_BLK, D), lambda i: (i, 0)),
            pl.BlockSpec((ROWS_BLK, D), lambda i: (i, 0)),
        ],
        out_shape=[
            jax.ShapeDtypeStruct((N, D), jnp.float32),
            jax.ShapeDtypeStruct((N, D), jnp.float32),
        ],
    )(x, W0, indeg2)


def _tc_mid(tmp, g, indeg2, b_row, W_next, batch_col):
    def body(tmp_ref, g_ref, deg_ref, b_ref, w_ref, bat_ref, gn_ref, gn2_ref, pool_ref):
        i = pl.program_id(0)
        dis = _dis_from(deg_ref)
        h = jnp.maximum(
            dis * (tmp_ref[0] + tmp_ref[1] + g_ref[...]) + b_ref[...], 0.0
        )
        oh = (bat_ref[...] == lax.broadcasted_iota(jnp.int32, (ROWS_BLK, NG), 1)).astype(
            jnp.float32
        )
        pc = lax.dot_general(
            oh, h, (((0,), (0,)), ((), ())), preferred_element_type=jnp.float32,
            precision=_P,
        )

        @pl.when(i == 0)
        def _():
            pool_ref[...] = pc

        @pl.when(i > 0)
        def _():
            pool_ref[...] += pc

        gn = dis * jnp.dot(
            h, w_ref[...], preferred_element_type=jnp.float32, precision=_P
        )
        gn_ref[...] = gn
        gn2_ref[...] = gn

    return pl.pallas_call(
        body,
        grid=(N // ROWS_BLK,),
        in_specs=[
            pl.BlockSpec((NC, ROWS_BLK, D), lambda i: (0, i, 0)),
            pl.BlockSpec((ROWS_BLK, D), lambda i: (i, 0)),
            pl.BlockSpec((NC, ROWS_BLK, D), lambda i: (0, i, 0)),
            pl.BlockSpec((1, D), lambda i: (0, 0)),
            pl.BlockSpec((D, D), lambda i: (0, 0)),
            pl.BlockSpec((ROWS_BLK, 1), lambda i: (i, 0)),
        ],
        out_specs=[
            pl.BlockSpec((ROWS_BLK, D), lambda i: (i, 0)),
            pl.BlockSpec((ROWS_BLK, D), lambda i: (i, 0)),
            pl.BlockSpec((NG, D), lambda i: (0, 0)),
        ],
        out_shape=[
            jax.ShapeDtypeStruct((N, D), jnp.float32),
            jax.ShapeDtypeStruct((N, D), jnp.float32),
            jax.ShapeDtypeStruct((NG, D), jnp.float32),
        ],
    )(tmp, g, indeg2, b_row, W_next, batch_col)


def _tc_final(tmp, g, indeg2, b_row, batch_col, lin1_W, lin1_b, lin2_W, lin2_b):
    nblk = N // ROWS_BLK

    def body(tmp_ref, g_ref, deg_ref, b_ref, bat_ref, w1_ref, b1_ref, w2_ref, b2_ref,
             pool_ref, out_ref):
        i = pl.program_id(0)
        dis = _dis_from(deg_ref)
        h = jnp.maximum(
            dis * (tmp_ref[0] + tmp_ref[1] + g_ref[...]) + b_ref[...], 0.0
        )
        oh = (bat_ref[...] == lax.broadcasted_iota(jnp.int32, (ROWS_BLK, NG), 1)).astype(
            jnp.float32
        )
        pc = lax.dot_general(
            oh, h, (((0,), (0,)), ((), ())), preferred_element_type=jnp.float32,
            precision=_P,
        )

        @pl.when(i == 0)
        def _():
            pool_ref[...] = pc

        @pl.when(i > 0)
        def _():
            pool_ref[...] += pc

        @pl.when(i == nblk - 1)
        def _():
            p = jnp.maximum(
                jnp.dot(pool_ref[...], w1_ref[...], preferred_element_type=jnp.float32,
                        precision=_P) + b1_ref[...],
                0.0,
            )
            out_ref[...] = jnp.dot(
                p, w2_ref[...], preferred_element_type=jnp.float32, precision=_P
            ) + b2_ref[...]

    return pl.pallas_call(
        body,
        grid=(nblk,),
        in_specs=[
            pl.BlockSpec((NC, ROWS_BLK, D), lambda i: (0, i, 0)),
            pl.BlockSpec((ROWS_BLK, D), lambda i: (i, 0)),
            pl.BlockSpec((NC, ROWS_BLK, D), lambda i: (0, i, 0)),
            pl.BlockSpec((1, D), lambda i: (0, 0)),
            pl.BlockSpec((ROWS_BLK, 1), lambda i: (i, 0)),
            pl.BlockSpec((D, D), lambda i: (0, 0)),
            pl.BlockSpec((1, D), lambda i: (0, 0)),
            pl.BlockSpec((D, NG), lambda i: (0, 0)),
            pl.BlockSpec((1, NG), lambda i: (0, 0)),
        ],
        out_specs=[
            pl.BlockSpec((NG, D), lambda i: (0, 0)),
            pl.BlockSpec((NG, NG), lambda i: (0, 0)),
        ],
        out_shape=[
            jax.ShapeDtypeStruct((NG, D), jnp.float32),
            jax.ShapeDtypeStruct((NG, NG), jnp.float32),
        ],
    )(tmp, g, indeg2, b_row, batch_col, lin1_W, lin1_b, lin2_W, lin2_b)


def kernel(x, edge_index, batch, W0, b0, W1, b1, W2, b2, lin1_W, lin1_b, lin2_W, lin2_b):
    src = edge_index[0]
    dst = edge_index[1]
    pad = EPAD - E
    pad_dst = N + (jnp.arange(pad, dtype=jnp.int32) % (ACC - N))
    dstp = jnp.concatenate([dst, pad_dst]).reshape(NT, NCHUNK, CHUNK)

    tot_chunks = NS * (NCH0 + NCH1)
    fpad = max(0, tot_chunks * CHUNK - E)
    fpad_dst = N + (jnp.arange(fpad, dtype=jnp.int32) % (ACC - N))
    fpad_src = jnp.arange(fpad, dtype=jnp.int32) % N
    src_f = jnp.concatenate([src, fpad_src])[: tot_chunks * CHUNK]
    dst_f = jnp.concatenate([dst, fpad_dst])[: tot_chunks * CHUNK]
    src_f = src_f.reshape(tot_chunks, CHUNK)
    dst_f = dst_f.reshape(tot_chunks, CHUNK)
    n0 = NS * NCH0
    edges0 = (src_f[:n0].reshape(NS, NCH0, CHUNK), dst_f[:n0].reshape(NS, NCH0, CHUNK))
    edges1 = (src_f[n0:].reshape(NS, NCH1, CHUNK), dst_f[n0:].reshape(NS, NCH1, CHUNK))
    zerosD = jnp.zeros((RPT, D), jnp.float32)
    onesD = jnp.ones((CHUNK, D), jnp.float32)
    batch_col = batch.reshape(N, 1)
    b0r = b0.reshape(1, D)
    b1r = b1.reshape(1, D)
    b2r = b2.reshape(1, D)
    lin1_br = lin1_b.reshape(1, D)
    lin2_br = lin2_b.reshape(1, NG)

    indeg2 = _sc_deg(dstp, onesD, zerosD)
    g1 = _tc_first(x, W0, indeg2)
    tmp1 = _sc_prop(g1, edges0, edges1, zerosD)
    g2a, g2b, pooled1 = _tc_mid(tmp1, g1[0], indeg2, b0r, W1, batch_col)
    tmp2 = _sc_prop((g2a, g2b), edges0, edges1, zerosD)
    g3a, g3b, pooled2 = _tc_mid(tmp2, g2a, indeg2, b1r, W2, batch_col)
    tmp3 = _sc_prop((g3a, g3b), edges0, edges1, zerosD)
    pooled3, out = _tc_final(
        tmp3, g3a, indeg2, b2r, batch_col, lin1_W, lin1_br, lin2_W, lin2_br
    )
    return (out, pooled1, pooled2, pooled3)


# speedup vs baseline: 1.4518x; 1.0123x over previous
import functools

import jax
import jax.numpy as jnp
from jax import lax
from jax.experimental import pallas as pl
from jax.experimental.pallas import tpu as pltpu
from jax.experimental.pallas import tpu_sc as plsc

N = 10000
E = 320000
D = 128
NG = 64

NC = 2
NS = 16
NT = NC * NS
CHUNK = 128
NCHUNK = 80
NB = 2
HC = 32
NCH0 = 80
NCH1 = 80
EPAD = NT * NCHUNK * CHUNK
ACC = 10240
RPT = ACC // NS

ROWS_BLK = 1000


def _mesh():
    return plsc.VectorSubcoreMesh(
        core_axis_name="c", subcore_axis_name="s", num_cores=NC, num_subcores=NS
    )


def _sc_deg(dst0, dst1, onesD, zerosD):
    nch_max = max(NCH0, NCH1)

    @functools.partial(
        pl.kernel,
        out_type=jax.ShapeDtypeStruct((NC, ACC, D), jnp.float32),
        mesh=_mesh(),
        scratch_types=[
            pltpu.VMEM((nch_max, CHUNK), jnp.int32),
            pltpu.VMEM((CHUNK, D), jnp.float32),
            pltpu.VMEM_SHARED((ACC, D), jnp.float32),
            pltpu.SemaphoreType.DMA,
        ],
    )
    def k(dst0_hbm, dst1_hbm, ones_hbm, zeros_hbm, out_hbm, idx_v, ones_v, acc_s, sem):
        c = lax.axis_index("c")
        s = lax.axis_index("s")
        pltpu.sync_copy(zeros_hbm, acc_s.at[pl.ds(s * RPT, RPT)])
        pltpu.sync_copy(ones_hbm, ones_v)
        plsc.subcore_barrier()

        def run(dst_hbm, nch):
            pltpu.sync_copy(dst_hbm.at[s], idx_v.at[pl.ds(0, nch)])

            def body(j, carry):
                pltpu.sync_copy(ones_v, acc_s.at[idx_v.at[j]], add=True)
                return carry

            lax.fori_loop(0, nch, body, 0)

        @pl.when(c == 0)
        def _():
            run(dst0_hbm, NCH0)

        @pl.when(c == 1)
        def _():
            run(dst1_hbm, NCH1)

        plsc.subcore_barrier()
        pltpu.sync_copy(acc_s.at[pl.ds(s * RPT, RPT)], out_hbm.at[c, pl.ds(s * RPT, RPT)])

    return k(dst0, dst1, onesD, zerosD)


def _sc_prop(g, edges0, edges1, zerosD):
    @functools.partial(
        pl.kernel,
        out_type=jax.ShapeDtypeStruct((NC, ACC, D), jnp.float32),
        mesh=_mesh(),
        scratch_types=[
            pltpu.VMEM((HC, CHUNK), jnp.int32),
            pltpu.VMEM((HC, CHUNK), jnp.int32),
            [pltpu.VMEM((CHUNK, D), jnp.float32)] * NB,
            pltpu.VMEM_SHARED((ACC, D), jnp.float32),
            pltpu.SemaphoreType.DMA,
        ],
    )
    def k(g_hbm, src0_hbm, dst0_hbm, src1_hbm, dst1_hbm, zeros_hbm, out_hbm,
          sidx, didx, rows, acc_s, gsem):
        c = lax.axis_index("c")
        s = lax.axis_index("s")
        pltpu.sync_copy(zeros_hbm, acc_s.at[pl.ds(s * RPT, RPT)])
        plsc.subcore_barrier()

        def slab(g_hbm, srcc_hbm, dstc_hbm, base, hc):
            pltpu.sync_copy(srcc_hbm.at[s, pl.ds(base, hc)], sidx.at[pl.ds(0, hc)])
            pltpu.sync_copy(dstc_hbm.at[s, pl.ds(base, hc)], didx.at[pl.ds(0, hc)])
            nprime = min(NB, hc)
            for b in range(nprime):
                pltpu.async_copy(g_hbm.at[sidx.at[b]], rows[b], gsem)

            def outer(jo, c2):
                jb = jo * NB
                for b in range(NB):
                    j = jb + b
                    pltpu.make_async_copy(g_hbm.at[sidx.at[j]], rows[b], gsem).wait()
                    pltpu.sync_copy(rows[b], acc_s.at[didx.at[j]], add=True)
                    nxt = j + NB

                    @pl.when(nxt < hc)
                    def _():
                        pltpu.async_copy(g_hbm.at[sidx.at[nxt]], rows[b], gsem)

                return c2

            lax.fori_loop(0, hc // NB, outer, 0)

        def run(g_hbm, srcc_hbm, dstc_hbm, nch):
            done = 0
            while done < nch:
                hc = min(HC, nch - done)
                slab(g_hbm, srcc_hbm, dstc_hbm, done, hc)
                done += hc

        @pl.when(c == 0)
        def _():
            run(g_hbm, src0_hbm, dst0_hbm, NCH0)

        @pl.when(c == 1)
        def _():
            run(g_hbm, src1_hbm, dst1_hbm, NCH1)

        plsc.subcore_barrier()
        pltpu.sync_copy(acc_s.at[pl.ds(s * RPT, RPT)], out_hbm.at[c, pl.ds(s * RPT, RPT)])

    return k(g, edges0[0], edges0[1], edges1[0], edges1[1], zerosD)


def _dis_from(deg_ref):
    return lax.rsqrt(deg_ref[0, :, 0:1] + deg_ref[1, :, 0:1] + 1.0)


_P = lax.Precision.HIGHEST


def _tc_first(x, W0, indeg2):
    def body(x_ref, w_ref, deg_ref, g_ref, dis_ref):
        dis = _dis_from(deg_ref)
        g_ref[...] = dis * jnp.dot(
            x_ref[...], w_ref[...], preferred_element_type=jnp.float32, precision=_P
        )
        dis_ref[...] = dis

    return pl.pallas_call(
        body,
        grid=(N // ROWS_BLK,),
        in_specs=[
            pl.BlockSpec((ROWS_BLK, D), lambda i: (i, 0)),
            pl.BlockSpec((D, D), lambda i: (0, 0)),
            pl.BlockSpec((NC, ROWS_BLK, D), lambda i: (0, i, 0)),
        ],
        out_specs=[
            pl.BlockSpec((ROWS_BLK, D), lambda i: (i, 0)),
            pl.BlockSpec((ROWS_BLK, 1), lambda i: (i, 0)),
        ],
        out_shape=[
            jax.ShapeDtypeStruct((N, D), jnp.float32),
            jax.ShapeDtypeStruct((N, 1), jnp.float32),
        ],
    )(x, W0, indeg2)


def _tc_mid(tmp, g, dis_col, b_row, W_next, batch_col):
    def body(tmp_ref, g_ref, dis_ref, b_ref, w_ref, bat_ref, gn_ref, pool_ref):
        i = pl.program_id(0)
        dis = dis_ref[...]
        h = jnp.maximum(
            dis * (tmp_ref[0] + tmp_ref[1] + g_ref[...]) + b_ref[...], 0.0
        )
        oh = (bat_ref[...] == lax.broadcasted_iota(jnp.int32, (ROWS_BLK, NG), 1)).astype(
            jnp.float32
        )
        pc = lax.dot_general(
            oh, h, (((0,), (0,)), ((), ())), preferred_element_type=jnp.float32,
            precision=_P,
        )

        @pl.when(i == 0)
        def _():
            pool_ref[...] = pc

        @pl.when(i > 0)
        def _():
            pool_ref[...] += pc

        gn_ref[...] = dis * jnp.dot(
            h, w_ref[...], preferred_element_type=jnp.float32, precision=_P
        )

    return pl.pallas_call(
        body,
        grid=(N // ROWS_BLK,),
        in_specs=[
            pl.BlockSpec((NC, ROWS_BLK, D), lambda i: (0, i, 0)),
            pl.BlockSpec((ROWS_BLK, D), lambda i: (i, 0)),
            pl.BlockSpec((ROWS_BLK, 1), lambda i: (i, 0)),
            pl.BlockSpec((1, D), lambda i: (0, 0)),
            pl.BlockSpec((D, D), lambda i: (0, 0)),
            pl.BlockSpec((ROWS_BLK, 1), lambda i: (i, 0)),
        ],
        out_specs=[
            pl.BlockSpec((ROWS_BLK, D), lambda i: (i, 0)),
            pl.BlockSpec((NG, D), lambda i: (0, 0)),
        ],
        out_shape=[
            jax.ShapeDtypeStruct((N, D), jnp.float32),
            jax.ShapeDtypeStruct((NG, D), jnp.float32),
        ],
    )(tmp, g, dis_col, b_row, W_next, batch_col)


def _tc_final(tmp, g, dis_col, b_row, batch_col, lin1_W, lin1_b, lin2_W, lin2_b):
    nblk = N // ROWS_BLK

    def body(tmp_ref, g_ref, dis_ref, b_ref, bat_ref, w1_ref, b1_ref, w2_ref, b2_ref,
             pool_ref, out_ref):
        i = pl.program_id(0)
        dis = dis_ref[...]
        h = jnp.maximum(
            dis * (tmp_ref[0] + tmp_ref[1] + g_ref[...]) + b_ref[...], 0.0
        )
        oh = (bat_ref[...] == lax.broadcasted_iota(jnp.int32, (ROWS_BLK, NG), 1)).astype(
            jnp.float32
        )
        pc = lax.dot_general(
            oh, h, (((0,), (0,)), ((), ())), preferred_element_type=jnp.float32,
            precision=_P,
        )

        @pl.when(i == 0)
        def _():
            pool_ref[...] = pc

        @pl.when(i > 0)
        def _():
            pool_ref[...] += pc

        @pl.when(i == nblk - 1)
        def _():
            p = jnp.maximum(
                jnp.dot(pool_ref[...], w1_ref[...], preferred_element_type=jnp.float32,
                        precision=_P) + b1_ref[...],
                0.0,
            )
            out_ref[...] = jnp.dot(
                p, w2_ref[...], preferred_element_type=jnp.float32, precision=_P
            ) + b2_ref[...]

    return pl.pallas_call(
        body,
        grid=(nblk,),
        in_specs=[
            pl.BlockSpec((NC, ROWS_BLK, D), lambda i: (0, i, 0)),
            pl.BlockSpec((ROWS_BLK, D), lambda i: (i, 0)),
            pl.BlockSpec((ROWS_BLK, 1), lambda i: (i, 0)),
            pl.BlockSpec((1, D), lambda i: (0, 0)),
            pl.BlockSpec((ROWS_BLK, 1), lambda i: (i, 0)),
            pl.BlockSpec((D, D), lambda i: (0, 0)),
            pl.BlockSpec((1, D), lambda i: (0, 0)),
            pl.BlockSpec((D, NG), lambda i: (0, 0)),
            pl.BlockSpec((1, NG), lambda i: (0, 0)),
        ],
        out_specs=[
            pl.BlockSpec((NG, D), lambda i: (0, 0)),
            pl.BlockSpec((NG, NG), lambda i: (0, 0)),
        ],
        out_shape=[
            jax.ShapeDtypeStruct((NG, D), jnp.float32),
            jax.ShapeDtypeStruct((NG, NG), jnp.float32),
        ],
    )(tmp, g, dis_col, b_row, batch_col, lin1_W, lin1_b, lin2_W, lin2_b)


def kernel(x, edge_index, batch, W0, b0, W1, b1, W2, b2, lin1_W, lin1_b, lin2_W, lin2_b):
    src = edge_index[0]
    dst = edge_index[1]
    tot_chunks = NS * (NCH0 + NCH1)
    fpad = max(0, tot_chunks * CHUNK - E)
    fpad_dst = N + (jnp.arange(fpad, dtype=jnp.int32) % (ACC - N))
    fpad_src = jnp.arange(fpad, dtype=jnp.int32) % N
    src_f = jnp.concatenate([src, fpad_src])[: tot_chunks * CHUNK]
    dst_f = jnp.concatenate([dst, fpad_dst])[: tot_chunks * CHUNK]
    src_f = src_f.reshape(tot_chunks, CHUNK)
    dst_f = dst_f.reshape(tot_chunks, CHUNK)
    n0 = NS * NCH0
    edges0 = (src_f[:n0].reshape(NS, NCH0, CHUNK), dst_f[:n0].reshape(NS, NCH0, CHUNK))
    edges1 = (src_f[n0:].reshape(NS, NCH1, CHUNK), dst_f[n0:].reshape(NS, NCH1, CHUNK))
    zerosD = jnp.zeros((RPT, D), jnp.float32)
    onesD = jnp.ones((CHUNK, D), jnp.float32)
    batch_col = batch.reshape(N, 1)
    b0r = b0.reshape(1, D)
    b1r = b1.reshape(1, D)
    b2r = b2.reshape(1, D)
    lin1_br = lin1_b.reshape(1, D)
    lin2_br = lin2_b.reshape(1, NG)

    indeg2 = _sc_deg(edges0[1], edges1[1], onesD, zerosD)
    g1, dis_col = _tc_first(x, W0, indeg2)
    tmp1 = _sc_prop(g1, edges0, edges1, zerosD)
    g2, pooled1 = _tc_mid(tmp1, g1, dis_col, b0r, W1, batch_col)
    tmp2 = _sc_prop(g2, edges0, edges1, zerosD)
    g3, pooled2 = _tc_mid(tmp2, g2, dis_col, b1r, W2, batch_col)
    tmp3 = _sc_prop(g3, edges0, edges1, zerosD)
    pooled3, out = _tc_final(
        tmp3, g3, dis_col, b2r, batch_col, lin1_W, lin1_br, lin2_W, lin2_br
    )
    return (out, pooled1, pooled2, pooled3)


# speedup vs baseline: 1.4538x; 1.0014x over previous
import functools

import jax
import jax.numpy as jnp
from jax import lax
from jax.experimental import pallas as pl
from jax.experimental.pallas import tpu as pltpu
from jax.experimental.pallas import tpu_sc as plsc

N = 10000
E = 320000
D = 128
NG = 64

NC = 2
NS = 16
NT = NC * NS
CHUNK = 128
NCHUNK = 80
NB = 2
HC = 32
NCH0 = 80
NCH1 = 80
EPAD = NT * NCHUNK * CHUNK
ACC = 10240
RPT = ACC // NS

ROWS_BLK = 1000


def _mesh():
    return plsc.VectorSubcoreMesh(
        core_axis_name="c", subcore_axis_name="s", num_cores=NC, num_subcores=NS
    )


def _sc_deg(dst0, dst1, onesD, zerosD):
    nch_max = max(NCH0, NCH1)

    @functools.partial(
        pl.kernel,
        out_type=jax.ShapeDtypeStruct((NC, ACC, D), jnp.float32),
        mesh=_mesh(),
        scratch_types=[
            pltpu.VMEM((nch_max, CHUNK), jnp.int32),
            pltpu.VMEM((CHUNK, D), jnp.float32),
            pltpu.VMEM_SHARED((ACC, D), jnp.float32),
            pltpu.SemaphoreType.DMA,
        ],
    )
    def k(dst0_hbm, dst1_hbm, ones_hbm, zeros_hbm, out_hbm, idx_v, ones_v, acc_s, sem):
        c = lax.axis_index("c")
        s = lax.axis_index("s")
        pltpu.sync_copy(zeros_hbm, acc_s.at[pl.ds(s * RPT, RPT)])
        pltpu.sync_copy(ones_hbm, ones_v)
        plsc.subcore_barrier()

        def run(dst_hbm, nch):
            pltpu.sync_copy(dst_hbm.at[s], idx_v.at[pl.ds(0, nch)])

            def fire(j, carry):
                pltpu.async_copy(ones_v, acc_s.at[idx_v.at[j]], sem, add=True)
                return carry

            lax.fori_loop(0, nch, fire, 0)

            def drain(j, carry):
                pltpu.make_async_copy(ones_v, acc_s.at[idx_v.at[j]], sem).wait()
                return carry

            lax.fori_loop(0, nch, drain, 0)

        @pl.when(c == 0)
        def _():
            run(dst0_hbm, NCH0)

        @pl.when(c == 1)
        def _():
            run(dst1_hbm, NCH1)

        plsc.subcore_barrier()
        pltpu.sync_copy(acc_s.at[pl.ds(s * RPT, RPT)], out_hbm.at[c, pl.ds(s * RPT, RPT)])

    return k(dst0, dst1, onesD, zerosD)


def _sc_prop(g, edges0, edges1, zerosD):
    @functools.partial(
        pl.kernel,
        out_type=jax.ShapeDtypeStruct((NC, ACC, D), jnp.float32),
        mesh=_mesh(),
        scratch_types=[
            pltpu.VMEM((HC, CHUNK), jnp.int32),
            pltpu.VMEM((HC, CHUNK), jnp.int32),
            [pltpu.VMEM((CHUNK, D), jnp.float32)] * NB,
            pltpu.VMEM_SHARED((ACC, D), jnp.float32),
            pltpu.SemaphoreType.DMA,
        ],
    )
    def k(g_hbm, src0_hbm, dst0_hbm, src1_hbm, dst1_hbm, zeros_hbm, out_hbm,
          sidx, didx, rows, acc_s, gsem):
        c = lax.axis_index("c")
        s = lax.axis_index("s")
        pltpu.sync_copy(zeros_hbm, acc_s.at[pl.ds(s * RPT, RPT)])
        plsc.subcore_barrier()

        def slab(g_hbm, srcc_hbm, dstc_hbm, base, hc):
            pltpu.sync_copy(srcc_hbm.at[s, pl.ds(base, hc)], sidx.at[pl.ds(0, hc)])
            pltpu.sync_copy(dstc_hbm.at[s, pl.ds(base, hc)], didx.at[pl.ds(0, hc)])
            nprime = min(NB, hc)
            for b in range(nprime):
                pltpu.async_copy(g_hbm.at[sidx.at[b]], rows[b], gsem)

            def outer(jo, c2):
                jb = jo * NB
                for b in range(NB):
                    j = jb + b
                    pltpu.make_async_copy(g_hbm.at[sidx.at[j]], rows[b], gsem).wait()
                    pltpu.sync_copy(rows[b], acc_s.at[didx.at[j]], add=True)
                    nxt = j + NB

                    @pl.when(nxt < hc)
                    def _():
                        pltpu.async_copy(g_hbm.at[sidx.at[nxt]], rows[b], gsem)

                return c2

            lax.fori_loop(0, hc // NB, outer, 0)

        def run(g_hbm, srcc_hbm, dstc_hbm, nch):
            done = 0
            while done < nch:
                hc = min(HC, nch - done)
                slab(g_hbm, srcc_hbm, dstc_hbm, done, hc)
                done += hc

        @pl.when(c == 0)
        def _():
            run(g_hbm, src0_hbm, dst0_hbm, NCH0)

        @pl.when(c == 1)
        def _():
            run(g_hbm, src1_hbm, dst1_hbm, NCH1)

        plsc.subcore_barrier()
        pltpu.sync_copy(acc_s.at[pl.ds(s * RPT, RPT)], out_hbm.at[c, pl.ds(s * RPT, RPT)])

    return k(g, edges0[0], edges0[1], edges1[0], edges1[1], zerosD)


def _dis_from(deg_ref):
    return lax.rsqrt(deg_ref[0, :, 0:1] + deg_ref[1, :, 0:1] + 1.0)


_P = lax.Precision.HIGHEST


def _tc_first(x, W0, indeg2):
    def body(x_ref, w_ref, deg_ref, g_ref, dis_ref):
        dis = _dis_from(deg_ref)
        g_ref[...] = dis * jnp.dot(
            x_ref[...], w_ref[...], preferred_element_type=jnp.float32, precision=_P
        )
        dis_ref[...] = dis

    return pl.pallas_call(
        body,
        grid=(N // ROWS_BLK,),
        in_specs=[
            pl.BlockSpec((ROWS_BLK, D), lambda i: (i, 0)),
            pl.BlockSpec((D, D), lambda i: (0, 0)),
            pl.BlockSpec((NC, ROWS_BLK, D), lambda i: (0, i, 0)),
        ],
        out_specs=[
            pl.BlockSpec((ROWS_BLK, D), lambda i: (i, 0)),
            pl.BlockSpec((ROWS_BLK, 1), lambda i: (i, 0)),
        ],
        out_shape=[
            jax.ShapeDtypeStruct((N, D), jnp.float32),
            jax.ShapeDtypeStruct((N, 1), jnp.float32),
        ],
    )(x, W0, indeg2)


def _tc_mid(tmp, g, dis_col, b_row, W_next, batch_col):
    def body(tmp_ref, g_ref, dis_ref, b_ref, w_ref, bat_ref, gn_ref, pool_ref):
        i = pl.program_id(0)
        dis = dis_ref[...]
        h = jnp.maximum(
            dis * (tmp_ref[0] + tmp_ref[1] + g_ref[...]) + b_ref[...], 0.0
        )
        oh = (bat_ref[...] == lax.broadcasted_iota(jnp.int32, (ROWS_BLK, NG), 1)).astype(
            jnp.float32
        )
        pc = lax.dot_general(
            oh, h, (((0,), (0,)), ((), ())), preferred_element_type=jnp.float32,
            precision=_P,
        )

        @pl.when(i == 0)
        def _():
            pool_ref[...] = pc

        @pl.when(i > 0)
        def _():
            pool_ref[...] += pc

        gn_ref[...] = dis * jnp.dot(
            h, w_ref[...], preferred_element_type=jnp.float32, precision=_P
        )

    return pl.pallas_call(
        body,
        grid=(N // ROWS_BLK,),
        in_specs=[
            pl.BlockSpec((NC, ROWS_BLK, D), lambda i: (0, i, 0)),
            pl.BlockSpec((ROWS_BLK, D), lambda i: (i, 0)),
            pl.BlockSpec((ROWS_BLK, 1), lambda i: (i, 0)),
            pl.BlockSpec((1, D), lambda i: (0, 0)),
            pl.BlockSpec((D, D), lambda i: (0, 0)),
            pl.BlockSpec((ROWS_BLK, 1), lambda i: (i, 0)),
        ],
        out_specs=[
            pl.BlockSpec((ROWS_BLK, D), lambda i: (i, 0)),
            pl.BlockSpec((NG, D), lambda i: (0, 0)),
        ],
        out_shape=[
            jax.ShapeDtypeStruct((N, D), jnp.float32),
            jax.ShapeDtypeStruct((NG, D), jnp.float32),
        ],
    )(tmp, g, dis_col, b_row, W_next, batch_col)


def _tc_final(tmp, g, dis_col, b_row, batch_col, lin1_W, lin1_b, lin2_W, lin2_b):
    nblk = N // ROWS_BLK

    def body(tmp_ref, g_ref, dis_ref, b_ref, bat_ref, w1_ref, b1_ref, w2_ref, b2_ref,
             pool_ref, out_ref):
        i = pl.program_id(0)
        dis = dis_ref[...]
        h = jnp.maximum(
            dis * (tmp_ref[0] + tmp_ref[1] + g_ref[...]) + b_ref[...], 0.0
        )
        oh = (bat_ref[...] == lax.broadcasted_iota(jnp.int32, (ROWS_BLK, NG), 1)).astype(
            jnp.float32
        )
        pc = lax.dot_general(
            oh, h, (((0,), (0,)), ((), ())), preferred_element_type=jnp.float32,
            precision=_P,
        )

        @pl.when(i == 0)
        def _():
            pool_ref[...] = pc

        @pl.when(i > 0)
        def _():
            pool_ref[...] += pc

        @pl.when(i == nblk - 1)
        def _():
            p = jnp.maximum(
                jnp.dot(pool_ref[...], w1_ref[...], preferred_element_type=jnp.float32,
                        precision=_P) + b1_ref[...],
                0.0,
            )
            out_ref[...] = jnp.dot(
                p, w2_ref[...], preferred_element_type=jnp.float32, precision=_P
            ) + b2_ref[...]

    return pl.pallas_call(
        body,
        grid=(nblk,),
        in_specs=[
            pl.BlockSpec((NC, ROWS_BLK, D), lambda i: (0, i, 0)),
            pl.BlockSpec((ROWS_BLK, D), lambda i: (i, 0)),
            pl.BlockSpec((ROWS_BLK, 1), lambda i: (i, 0)),
            pl.BlockSpec((1, D), lambda i: (0, 0)),
            pl.BlockSpec((ROWS_BLK, 1), lambda i: (i, 0)),
            pl.BlockSpec((D, D), lambda i: (0, 0)),
            pl.BlockSpec((1, D), lambda i: (0, 0)),
            pl.BlockSpec((D, NG), lambda i: (0, 0)),
            pl.BlockSpec((1, NG), lambda i: (0, 0)),
        ],
        out_specs=[
            pl.BlockSpec((NG, D), lambda i: (0, 0)),
            pl.BlockSpec((NG, NG), lambda i: (0, 0)),
        ],
        out_shape=[
            jax.ShapeDtypeStruct((NG, D), jnp.float32),
            jax.ShapeDtypeStruct((NG, NG), jnp.float32),
        ],
    )(tmp, g, dis_col, b_row, batch_col, lin1_W, lin1_b, lin2_W, lin2_b)


def kernel(x, edge_index, batch, W0, b0, W1, b1, W2, b2, lin1_W, lin1_b, lin2_W, lin2_b):
    src = edge_index[0]
    dst = edge_index[1]
    tot_chunks = NS * (NCH0 + NCH1)
    fpad = max(0, tot_chunks * CHUNK - E)
    fpad_dst = N + (jnp.arange(fpad, dtype=jnp.int32) % (ACC - N))
    fpad_src = jnp.arange(fpad, dtype=jnp.int32) % N
    src_f = jnp.concatenate([src, fpad_src])[: tot_chunks * CHUNK]
    dst_f = jnp.concatenate([dst, fpad_dst])[: tot_chunks * CHUNK]
    src_f = src_f.reshape(tot_chunks, CHUNK)
    dst_f = dst_f.reshape(tot_chunks, CHUNK)
    n0 = NS * NCH0
    edges0 = (src_f[:n0].reshape(NS, NCH0, CHUNK), dst_f[:n0].reshape(NS, NCH0, CHUNK))
    edges1 = (src_f[n0:].reshape(NS, NCH1, CHUNK), dst_f[n0:].reshape(NS, NCH1, CHUNK))
    zerosD = jnp.zeros((RPT, D), jnp.float32)
    onesD = jnp.ones((CHUNK, D), jnp.float32)
    batch_col = batch.reshape(N, 1)
    b0r = b0.reshape(1, D)
    b1r = b1.reshape(1, D)
    b2r = b2.reshape(1, D)
    lin1_br = lin1_b.reshape(1, D)
    lin2_br = lin2_b.reshape(1, NG)

    indeg2 = _sc_deg(edges0[1], edges1[1], onesD, zerosD)
    g1, dis_col = _tc_first(x, W0, indeg2)
    tmp1 = _sc_prop(g1, edges0, edges1, zerosD)
    g2, pooled1 = _tc_mid(tmp1, g1, dis_col, b0r, W1, batch_col)
    tmp2 = _sc_prop(g2, edges0, edges1, zerosD)
    g3, pooled2 = _tc_mid(tmp2, g2, dis_col, b1r, W2, batch_col)
    tmp3 = _sc_prop(g3, edges0, edges1, zerosD)
    pooled3, out = _tc_final(
        tmp3, g3, dis_col, b2r, batch_col, lin1_W, lin1_br, lin2_W, lin2_br
    )
    return (out, pooled1, pooled2, pooled3)


# speedup vs baseline: 1.5208x; 1.0461x over previous
import functools

import jax
import jax.numpy as jnp
from jax import lax
from jax.experimental import pallas as pl
from jax.experimental.pallas import tpu as pltpu
from jax.experimental.pallas import tpu_sc as plsc

N = 10000
E = 320000
D = 128
NG = 64

NC = 2
NS = 16
NT = NC * NS
CHUNK = 128
NCHUNK = 80
NB = 2
HC = 32
NCH0 = 80
NCH1 = 80
EPAD = NT * NCHUNK * CHUNK
ACC = 10240
RPT = ACC // NS

ROWS_BLK = 2000


def _mesh():
    return plsc.VectorSubcoreMesh(
        core_axis_name="c", subcore_axis_name="s", num_cores=NC, num_subcores=NS
    )


def _sc_deg(dst0, dst1, onesD, zerosD):
    nch_max = max(NCH0, NCH1)

    @functools.partial(
        pl.kernel,
        out_type=jax.ShapeDtypeStruct((NC, ACC, D), jnp.float32),
        mesh=_mesh(),
        scratch_types=[
            pltpu.VMEM((nch_max, CHUNK), jnp.int32),
            pltpu.VMEM((CHUNK, D), jnp.float32),
            pltpu.VMEM_SHARED((ACC, D), jnp.float32),
            pltpu.SemaphoreType.DMA,
        ],
    )
    def k(dst0_hbm, dst1_hbm, ones_hbm, zeros_hbm, out_hbm, idx_v, ones_v, acc_s, sem):
        c = lax.axis_index("c")
        s = lax.axis_index("s")
        pltpu.sync_copy(zeros_hbm, acc_s.at[pl.ds(s * RPT, RPT)])
        pltpu.sync_copy(ones_hbm, ones_v)
        plsc.subcore_barrier()

        def run(dst_hbm, nch):
            pltpu.sync_copy(dst_hbm.at[s], idx_v.at[pl.ds(0, nch)])

            def fire(j, carry):
                pltpu.async_copy(ones_v, acc_s.at[idx_v.at[j]], sem, add=True)
                return carry

            lax.fori_loop(0, nch, fire, 0)

            def drain(j, carry):
                pltpu.make_async_copy(ones_v, acc_s.at[idx_v.at[j]], sem).wait()
                return carry

            lax.fori_loop(0, nch, drain, 0)

        @pl.when(c == 0)
        def _():
            run(dst0_hbm, NCH0)

        @pl.when(c == 1)
        def _():
            run(dst1_hbm, NCH1)

        plsc.subcore_barrier()
        pltpu.sync_copy(acc_s.at[pl.ds(s * RPT, RPT)], out_hbm.at[c, pl.ds(s * RPT, RPT)])

    return k(dst0, dst1, onesD, zerosD)


def _sc_prop(g, edges0, edges1, zerosD):
    @functools.partial(
        pl.kernel,
        out_type=jax.ShapeDtypeStruct((NC, ACC, D), jnp.float32),
        mesh=_mesh(),
        scratch_types=[
            pltpu.VMEM((HC, CHUNK), jnp.int32),
            pltpu.VMEM((HC, CHUNK), jnp.int32),
            [pltpu.VMEM((CHUNK, D), jnp.float32)] * NB,
            pltpu.VMEM_SHARED((ACC, D), jnp.float32),
            pltpu.SemaphoreType.DMA,
        ],
    )
    def k(g_hbm, src0_hbm, dst0_hbm, src1_hbm, dst1_hbm, zeros_hbm, out_hbm,
          sidx, didx, rows, acc_s, gsem):
        c = lax.axis_index("c")
        s = lax.axis_index("s")
        pltpu.sync_copy(zeros_hbm, acc_s.at[pl.ds(s * RPT, RPT)])
        plsc.subcore_barrier()

        def slab(g_hbm, srcc_hbm, dstc_hbm, base, hc):
            pltpu.sync_copy(srcc_hbm.at[s, pl.ds(base, hc)], sidx.at[pl.ds(0, hc)])
            pltpu.sync_copy(dstc_hbm.at[s, pl.ds(base, hc)], didx.at[pl.ds(0, hc)])
            nprime = min(NB, hc)
            for b in range(nprime):
                pltpu.async_copy(g_hbm.at[sidx.at[b]], rows[b], gsem)

            def outer(jo, c2):
                jb = jo * NB
                for b in range(NB):
                    j = jb + b
                    pltpu.make_async_copy(g_hbm.at[sidx.at[j]], rows[b], gsem).wait()
                    pltpu.sync_copy(rows[b], acc_s.at[didx.at[j]], add=True)
                    nxt = j + NB

                    @pl.when(nxt < hc)
                    def _():
                        pltpu.async_copy(g_hbm.at[sidx.at[nxt]], rows[b], gsem)

                return c2

            lax.fori_loop(0, hc // NB, outer, 0)

        def run(g_hbm, srcc_hbm, dstc_hbm, nch):
            done = 0
            while done < nch:
                hc = min(HC, nch - done)
                slab(g_hbm, srcc_hbm, dstc_hbm, done, hc)
                done += hc

        @pl.when(c == 0)
        def _():
            run(g_hbm, src0_hbm, dst0_hbm, NCH0)

        @pl.when(c == 1)
        def _():
            run(g_hbm, src1_hbm, dst1_hbm, NCH1)

        plsc.subcore_barrier()
        pltpu.sync_copy(acc_s.at[pl.ds(s * RPT, RPT)], out_hbm.at[c, pl.ds(s * RPT, RPT)])

    return k(g, edges0[0], edges0[1], edges1[0], edges1[1], zerosD)


def _dis_from(deg_ref):
    return lax.rsqrt(deg_ref[0, :, 0:1] + deg_ref[1, :, 0:1] + 1.0)


_P = lax.Precision.HIGHEST


def _tc_first(x, W0, indeg2):
    def body(x_ref, w_ref, deg_ref, g_ref, dis_ref):
        dis = _dis_from(deg_ref)
        g_ref[...] = dis * jnp.dot(
            x_ref[...], w_ref[...], preferred_element_type=jnp.float32, precision=_P
        )
        dis_ref[...] = dis

    return pl.pallas_call(
        body,
        grid=(N // ROWS_BLK,),
        in_specs=[
            pl.BlockSpec((ROWS_BLK, D), lambda i: (i, 0)),
            pl.BlockSpec((D, D), lambda i: (0, 0)),
            pl.BlockSpec((NC, ROWS_BLK, D), lambda i: (0, i, 0)),
        ],
        out_specs=[
            pl.BlockSpec((ROWS_BLK, D), lambda i: (i, 0)),
            pl.BlockSpec((ROWS_BLK, 1), lambda i: (i, 0)),
        ],
        out_shape=[
            jax.ShapeDtypeStruct((N, D), jnp.float32),
            jax.ShapeDtypeStruct((N, 1), jnp.float32),
        ],
    )(x, W0, indeg2)


def _tc_mid(tmp, g, dis_col, b_row, W_next, batch_col):
    def body(tmp_ref, g_ref, dis_ref, b_ref, w_ref, bat_ref, gn_ref, pool_ref):
        i = pl.program_id(0)
        dis = dis_ref[...]
        h = jnp.maximum(
            dis * (tmp_ref[0] + tmp_ref[1] + g_ref[...]) + b_ref[...], 0.0
        )
        oh = (bat_ref[...] == lax.broadcasted_iota(jnp.int32, (ROWS_BLK, NG), 1)).astype(
            jnp.float32
        )
        pc = lax.dot_general(
            oh, h, (((0,), (0,)), ((), ())), preferred_element_type=jnp.float32,
            precision=lax.Precision.DEFAULT,
        )

        @pl.when(i == 0)
        def _():
            pool_ref[...] = pc

        @pl.when(i > 0)
        def _():
            pool_ref[...] += pc

        gn_ref[...] = dis * jnp.dot(
            h, w_ref[...], preferred_element_type=jnp.float32, precision=_P
        )

    return pl.pallas_call(
        body,
        grid=(N // ROWS_BLK,),
        in_specs=[
            pl.BlockSpec((NC, ROWS_BLK, D), lambda i: (0, i, 0)),
            pl.BlockSpec((ROWS_BLK, D), lambda i: (i, 0)),
            pl.BlockSpec((ROWS_BLK, 1), lambda i: (i, 0)),
            pl.BlockSpec((1, D), lambda i: (0, 0)),
            pl.BlockSpec((D, D), lambda i: (0, 0)),
            pl.BlockSpec((ROWS_BLK, 1), lambda i: (i, 0)),
        ],
        out_specs=[
            pl.BlockSpec((ROWS_BLK, D), lambda i: (i, 0)),
            pl.BlockSpec((NG, D), lambda i: (0, 0)),
        ],
        out_shape=[
            jax.ShapeDtypeStruct((N, D), jnp.float32),
            jax.ShapeDtypeStruct((NG, D), jnp.float32),
        ],
    )(tmp, g, dis_col, b_row, W_next, batch_col)


def _tc_final(tmp, g, dis_col, b_row, batch_col, lin1_W, lin1_b, lin2_W, lin2_b):
    nblk = N // ROWS_BLK

    def body(tmp_ref, g_ref, dis_ref, b_ref, bat_ref, w1_ref, b1_ref, w2_ref, b2_ref,
             pool_ref, out_ref):
        i = pl.program_id(0)
        dis = dis_ref[...]
        h = jnp.maximum(
            dis * (tmp_ref[0] + tmp_ref[1] + g_ref[...]) + b_ref[...], 0.0
        )
        oh = (bat_ref[...] == lax.broadcasted_iota(jnp.int32, (ROWS_BLK, NG), 1)).astype(
            jnp.float32
        )
        pc = lax.dot_general(
            oh, h, (((0,), (0,)), ((), ())), preferred_element_type=jnp.float32,
            precision=lax.Precision.DEFAULT,
        )

        @pl.when(i == 0)
        def _():
            pool_ref[...] = pc

        @pl.when(i > 0)
        def _():
            pool_ref[...] += pc

        @pl.when(i == nblk - 1)
        def _():
            p = jnp.maximum(
                jnp.dot(pool_ref[...], w1_ref[...], preferred_element_type=jnp.float32,
                        precision=_P) + b1_ref[...],
                0.0,
            )
            out_ref[...] = jnp.dot(
                p, w2_ref[...], preferred_element_type=jnp.float32, precision=_P
            ) + b2_ref[...]

    return pl.pallas_call(
        body,
        grid=(nblk,),
        in_specs=[
            pl.BlockSpec((NC, ROWS_BLK, D), lambda i: (0, i, 0)),
            pl.BlockSpec((ROWS_BLK, D), lambda i: (i, 0)),
            pl.BlockSpec((ROWS_BLK, 1), lambda i: (i, 0)),
            pl.BlockSpec((1, D), lambda i: (0, 0)),
            pl.BlockSpec((ROWS_BLK, 1), lambda i: (i, 0)),
            pl.BlockSpec((D, D), lambda i: (0, 0)),
            pl.BlockSpec((1, D), lambda i: (0, 0)),
            pl.BlockSpec((D, NG), lambda i: (0, 0)),
            pl.BlockSpec((1, NG), lambda i: (0, 0)),
        ],
        out_specs=[
            pl.BlockSpec((NG, D), lambda i: (0, 0)),
            pl.BlockSpec((NG, NG), lambda i: (0, 0)),
        ],
        out_shape=[
            jax.ShapeDtypeStruct((NG, D), jnp.float32),
            jax.ShapeDtypeStruct((NG, NG), jnp.float32),
        ],
    )(tmp, g, dis_col, b_row, batch_col, lin1_W, lin1_b, lin2_W, lin2_b)


def kernel(x, edge_index, batch, W0, b0, W1, b1, W2, b2, lin1_W, lin1_b, lin2_W, lin2_b):
    src = edge_index[0]
    dst = edge_index[1]
    tot_chunks = NS * (NCH0 + NCH1)
    fpad = max(0, tot_chunks * CHUNK - E)
    fpad_dst = N + (jnp.arange(fpad, dtype=jnp.int32) % (ACC - N))
    fpad_src = jnp.arange(fpad, dtype=jnp.int32) % N
    src_f = jnp.concatenate([src, fpad_src])[: tot_chunks * CHUNK]
    dst_f = jnp.concatenate([dst, fpad_dst])[: tot_chunks * CHUNK]
    src_f = src_f.reshape(tot_chunks, CHUNK)
    dst_f = dst_f.reshape(tot_chunks, CHUNK)
    n0 = NS * NCH0
    edges0 = (src_f[:n0].reshape(NS, NCH0, CHUNK), dst_f[:n0].reshape(NS, NCH0, CHUNK))
    edges1 = (src_f[n0:].reshape(NS, NCH1, CHUNK), dst_f[n0:].reshape(NS, NCH1, CHUNK))
    zerosD = jnp.zeros((RPT, D), jnp.float32)
    onesD = jnp.ones((CHUNK, D), jnp.float32)
    batch_col = batch.reshape(N, 1)
    b0r = b0.reshape(1, D)
    b1r = b1.reshape(1, D)
    b2r = b2.reshape(1, D)
    lin1_br = lin1_b.reshape(1, D)
    lin2_br = lin2_b.reshape(1, NG)

    indeg2 = _sc_deg(edges0[1], edges1[1], onesD, zerosD)
    g1, dis_col = _tc_first(x, W0, indeg2)
    tmp1 = _sc_prop(g1, edges0, edges1, zerosD)
    g2, pooled1 = _tc_mid(tmp1, g1, dis_col, b0r, W1, batch_col)
    tmp2 = _sc_prop(g2, edges0, edges1, zerosD)
    g3, pooled2 = _tc_mid(tmp2, g2, dis_col, b1r, W2, batch_col)
    tmp3 = _sc_prop(g3, edges0, edges1, zerosD)
    pooled3, out = _tc_final(
        tmp3, g3, dis_col, b2r, batch_col, lin1_W, lin1_br, lin2_W, lin2_br
    )
    return (out, pooled1, pooled2, pooled3)
